# Initial kernel scaffold; baseline (speedup 1.0000x reference)
import jax
import jax.numpy as jnp
from jax.experimental import pallas as pl


def kernel(x, edge_index, batch, Wl0, Wr0, att0, b0, Wl1, Wr1, att1, b1, Wl2, Wr2, att2, b2, lin_W, lin_b, out_W, out_b):
    raise NotImplementedError("write your pallas kernel here")


# speedup vs baseline: 7.9505x; 7.9505x over previous
import functools

import jax
import jax.numpy as jnp
from jax import lax
from jax.experimental import pallas as pl
from jax.experimental.pallas import tpu as pltpu
from jax.experimental.pallas import tpu_sc as plsc

N = 10000
E = 320000
G = 256
D = 128

NP = 10112
SENT = 10008
K = 48
NC = 2
NS = 16
NW = NC * NS
NCH = 216
EPW = NCH * K
ETOT = NW * EPW
RPT = NP // NS
BR = 2528
TINY = 1e-30


def _gat_sc_body(xl_hbm, xr_hbm, eidx_hbm, att_hbm,
                 out_acc, out_den,
                 idx_v, bufL, bufR, stage_p, den_v, alpha_sm, att_v,
                 accum_sh, semL, semR, semI0, semI1, semI2):
    c = lax.axis_index("c")
    s = lax.axis_index("s")
    wid = c * NS + s
    base = wid * NCH * 2 * K

    pltpu.sync_copy(att_hbm, att_v)

    def _zrow(e, _):
        for k in range(D // 16):
            bufL[0, e, pl.ds(k * 16, 16)] = jnp.zeros((16,), jnp.float32)
        stage_p[e, :] = jnp.zeros((16,), jnp.float32)
        return 0
    lax.fori_loop(0, K, _zrow, 0)

    def _zden(i, _):
        den_v[0, pl.ds(i * 16, 16)] = jnp.zeros((16,), jnp.float32)
        return 0
    lax.fori_loop(0, NP // 16, _zden, 0)

    r0 = s * RPT
    for t in range(RPT // K):
        pltpu.sync_copy(bufL.at[0], accum_sh.at[pl.ds(r0 + t * K, K)])
    rem = RPT - (RPT // K) * K
    pltpu.sync_copy(bufL.at[0, pl.ds(0, rem)],
                    accum_sh.at[pl.ds(r0 + (RPT // K) * K, rem)])
    plsc.subcore_barrier()

    iota16 = lax.iota(jnp.int32, 16)
    zz16 = jnp.zeros((16,), jnp.int32)

    def _chunk(j, _):
        b = 0
        slot = 0
        off = base + j * 2 * K
        pltpu.sync_copy(eidx_hbm.at[pl.ds(off, K)], idx_v.at[slot, 0])
        pltpu.sync_copy(eidx_hbm.at[pl.ds(off + K, K)], idx_v.at[slot, 1])
        pltpu.async_copy(xl_hbm.at[idx_v.at[slot, 0]], bufL.at[b], semL).wait()
        pltpu.async_copy(xr_hbm.at[idx_v.at[slot, 1]], bufR.at[b], semR).wait()

        def _alpha(e, _):
            sacc = jnp.zeros((16,), jnp.float32)
            for k in range(D // 16):
                l = bufL[b, e, pl.ds(k * 16, 16)]
                r = bufR[b, e, pl.ds(k * 16, 16)]
                z = l + r
                z = jnp.maximum(z, 0.2 * z)
                sacc = sacc + z * att_v[pl.ds(k * 16, 16)]
            alpha_sm[e] = jnp.sum(sacc)
            return 0
        lax.fori_loop(0, K, _alpha, 0)

        def _scale(e, _):
            pv = jnp.exp(jnp.full((16,), alpha_sm[e], jnp.float32))
            for k in range(D // 16):
                bufL[b, e, pl.ds(k * 16, 16)] = bufL[b, e, pl.ds(k * 16, 16)] * pv
            stage_p[e, :] = pv
            return 0
        lax.fori_loop(0, K, _scale, 0)

        for g in range(K // 16):
            p16 = plsc.load_gather(stage_p, [g * 16 + iota16, iota16])
            dstv = idx_v[slot, 1, pl.ds(g * 16, 16)]
            plsc.addupdate_scatter(den_v, [zz16, dstv], p16)

        pltpu.sync_copy(bufL.at[b], accum_sh.at[idx_v.at[slot, 1]], add=True)
        return 0

    lax.fori_loop(0, NCH, _chunk, 0)

    plsc.subcore_barrier()
    pltpu.sync_copy(accum_sh.at[pl.ds(r0, RPT)],
                    out_acc.at[pl.ds(c * NP + r0, RPT)])
    pltpu.sync_copy(den_v, out_den.at[wid])


_gat_sc = functools.partial(
    pl.kernel,
    out_type=[
        jax.ShapeDtypeStruct((NC * NP, D), jnp.float32),
        jax.ShapeDtypeStruct((NW, 1, NP), jnp.float32),
    ],
    mesh=plsc.VectorSubcoreMesh(core_axis_name="c", subcore_axis_name="s"),
    compiler_params=pltpu.CompilerParams(needs_layout_passes=False),
    scratch_types=[
        pltpu.VMEM((3, 2, K), jnp.int32),
        pltpu.VMEM((2, K, D), jnp.float32),
        pltpu.VMEM((2, K, D), jnp.float32),
        pltpu.VMEM((K, 16), jnp.float32),
        pltpu.VMEM((1, NP), jnp.float32),
        pltpu.SMEM((K,), jnp.float32),
        pltpu.VMEM((D,), jnp.float32),
        pltpu.VMEM_SHARED((NP, D), jnp.float32),
        pltpu.SemaphoreType.DMA,
        pltpu.SemaphoreType.DMA,
        pltpu.SemaphoreType.DMA,
        pltpu.SemaphoreType.DMA,
        pltpu.SemaphoreType.DMA,
    ],
)(_gat_sc_body)


def _mm2_body(x_ref, wl_ref, wr_ref, xl_ref, xr_ref):
    xv = x_ref[...]
    xl_ref[...] = jnp.dot(xv, wl_ref[...], preferred_element_type=jnp.float32)
    xr_ref[...] = jnp.dot(xv, wr_ref[...], preferred_element_type=jnp.float32)


_mm2 = pl.pallas_call(
    _mm2_body,
    grid=(NP // BR,),
    in_specs=[
        pl.BlockSpec((BR, D), lambda i: (i, 0)),
        pl.BlockSpec((D, D), lambda i: (0, 0)),
        pl.BlockSpec((D, D), lambda i: (0, 0)),
    ],
    out_specs=[pl.BlockSpec((BR, D), lambda i: (i, 0))] * 2,
    out_shape=[jax.ShapeDtypeStruct((NP, D), jnp.float32)] * 2,
)


def _combine_mm_body(acc_ref, den_ref, b_ref, wl_ref, wr_ref, xl_ref, xr_ref):
    a = acc_ref[0] + acc_ref[1]
    d = jnp.sum(den_ref[...], axis=1)[:, None]
    h = jnp.maximum(a / jnp.maximum(d, TINY) + b_ref[...], 0.0)
    xl_ref[...] = jnp.dot(h, wl_ref[...], preferred_element_type=jnp.float32)
    xr_ref[...] = jnp.dot(h, wr_ref[...], preferred_element_type=jnp.float32)


_combine_mm = pl.pallas_call(
    _combine_mm_body,
    grid=(NP // BR,),
    in_specs=[
        pl.BlockSpec((NC, BR, D), lambda i: (0, i, 0)),
        pl.BlockSpec((BR, NW), lambda i: (i, 0)),
        pl.BlockSpec((1, D), lambda i: (0, 0)),
        pl.BlockSpec((D, D), lambda i: (0, 0)),
        pl.BlockSpec((D, D), lambda i: (0, 0)),
    ],
    out_specs=[pl.BlockSpec((BR, D), lambda i: (i, 0))] * 2,
    out_shape=[jax.ShapeDtypeStruct((NP, D), jnp.float32)] * 2,
)


def _head_body(acc_ref, den_ref, b_ref, batch_ref, linW_ref, linb_ref,
               outW_ref, outb_ref, o_ref):
    a = acc_ref[0] + acc_ref[1]
    d = jnp.sum(den_ref[...], axis=1)[:, None]
    h = jnp.maximum(a / jnp.maximum(d, TINY) + b_ref[...], 0.0)
    gids = lax.broadcasted_iota(jnp.int32, (G, NP), 0)
    onehot = (batch_ref[...] == gids).astype(jnp.float32)
    ssum = jnp.dot(onehot, h, preferred_element_type=jnp.float32)
    cnt = jnp.sum(onehot, axis=1, keepdims=True)
    pooled = ssum / jnp.maximum(cnt, 1.0)
    t = jnp.maximum(
        jnp.dot(pooled, linW_ref[...], preferred_element_type=jnp.float32)
        + linb_ref[...], 0.0)
    o_ref[...] = (jnp.dot(t, outW_ref[...], preferred_element_type=jnp.float32)
                  + outb_ref[...])


_head = pl.pallas_call(
    _head_body,
    out_shape=jax.ShapeDtypeStruct((G, 1), jnp.float32),
)


def kernel(x, edge_index, batch, Wl0, Wr0, att0, b0, Wl1, Wr1, att1, b1,
           Wl2, Wr2, att2, b2, lin_W, lin_b, out_W, out_b):
    f32 = jnp.float32
    x_p = jnp.zeros((NP, D), f32).at[:N].set(x)
    loops = jnp.arange(N, dtype=jnp.int32)
    pad = jnp.full((ETOT - E - N,), SENT, jnp.int32)
    srcp = jnp.concatenate([edge_index[0], loops, pad]).reshape(NW, NCH, 1, K)
    dstp = jnp.concatenate([edge_index[1], loops, pad]).reshape(NW, NCH, 1, K)
    eidx = jnp.concatenate([srcp, dstp], axis=2).reshape(-1)
    batch_p = jnp.concatenate(
        [batch, jnp.full((NP - N,), G, jnp.int32)]).reshape(1, NP)

    def layer(xl, xr, att):
        acc, den = _gat_sc(xl, xr, eidx, att)
        return acc.reshape(NC, NP, D), den.reshape(NW, NP).T

    xl, xr = _mm2(x_p, Wl0, Wr0)
    acc, den = layer(xl, xr, att0)
    xl, xr = _combine_mm(acc, den, b0.reshape(1, D), Wl1, Wr1)
    acc, den = layer(xl, xr, att1)
    xl, xr = _combine_mm(acc, den, b1.reshape(1, D), Wl2, Wr2)
    acc, den = layer(xl, xr, att2)
    return _head(acc, den, b2.reshape(1, D), batch_p,
                 lin_W, lin_b.reshape(1, D // 2), out_W, out_b.reshape(1, 1))


# speedup vs baseline: 19.5822x; 2.4630x over previous
import functools

import jax
import jax.numpy as jnp
from jax import lax
from jax.experimental import pallas as pl
from jax.experimental.pallas import tpu as pltpu
from jax.experimental.pallas import tpu_sc as plsc

N = 10000
E = 320000
G = 256
D = 128

NP = 10112
SENT = 10008
K = 48
NC = 2
NS = 16
NW = NC * NS
NCH = 216
EPW = NCH * K
ETOT = NW * EPW
RPT = NP // NS
BR = 2528
TINY = 1e-30


def _gat_sc_body(xl_hbm, xr_hbm, eidx_hbm, att_hbm,
                 out_acc, out_den,
                 idx_v, bufL, bufR, stage_p, den_v, alpha_sm, att_v,
                 accum_sh, semL, semR, semI0, semI1, semI2):
    c = lax.axis_index("c")
    s = lax.axis_index("s")
    wid = c * NS + s
    base = wid * NCH * 2 * K

    pltpu.sync_copy(att_hbm, att_v)

    def _zrow(e, _):
        for k in range(D // 16):
            bufL[0, e, pl.ds(k * 16, 16)] = jnp.zeros((16,), jnp.float32)
        stage_p[e, :] = jnp.zeros((16,), jnp.float32)
        return 0
    lax.fori_loop(0, K, _zrow, 0)

    def _zden(i, _):
        den_v[0, pl.ds(i * 16, 16)] = jnp.zeros((16,), jnp.float32)
        return 0
    lax.fori_loop(0, NP // 16, _zden, 0)

    r0 = s * RPT
    for t in range(RPT // K):
        pltpu.sync_copy(bufL.at[0], accum_sh.at[pl.ds(r0 + t * K, K)])
    rem = RPT - (RPT // K) * K
    pltpu.sync_copy(bufL.at[0, pl.ds(0, rem)],
                    accum_sh.at[pl.ds(r0 + (RPT // K) * K, rem)])
    plsc.subcore_barrier()

    iota16 = lax.iota(jnp.int32, 16)
    zz16 = jnp.zeros((16,), jnp.int32)
    sem_idx = (semI0, semI1, semI2)

    def _idx_load(j, slot):
        off = base + j * 2 * K
        pltpu.async_copy(eidx_hbm.at[pl.ds(off, K)], idx_v.at[slot, 0],
                         sem_idx[slot])
        pltpu.async_copy(eidx_hbm.at[pl.ds(off + K, K)], idx_v.at[slot, 1],
                         sem_idx[slot])

    def _idx_wait(slot):
        pltpu.make_async_copy(eidx_hbm.at[pl.ds(0, K)], idx_v.at[slot, 0],
                              sem_idx[slot]).wait()
        pltpu.make_async_copy(eidx_hbm.at[pl.ds(0, K)], idx_v.at[slot, 1],
                              sem_idx[slot]).wait()

    for j0 in range(3):
        _idx_load(j0, j0)
    _idx_wait(0)
    pltpu.async_copy(xl_hbm.at[idx_v.at[0, 0]], bufL.at[0], semL)
    pltpu.async_copy(xr_hbm.at[idx_v.at[0, 1]], bufR.at[0], semR)

    def _chunk(j, u):
        b = u % 2
        slot = u % 3
        nslot = (u + 1) % 3
        pltpu.make_async_copy(xl_hbm.at[idx_v.at[slot, 0]], bufL.at[b],
                              semL).wait()
        pltpu.make_async_copy(xr_hbm.at[idx_v.at[slot, 1]], bufR.at[b],
                              semR).wait()

        @pl.when(j + 1 < NCH)
        def _():
            _idx_wait(nslot)
            pltpu.async_copy(xl_hbm.at[idx_v.at[nslot, 0]], bufL.at[1 - b],
                             semL)
            pltpu.async_copy(xr_hbm.at[idx_v.at[nslot, 1]], bufR.at[1 - b],
                             semR)

        def _alpha(e2, _):
            for d in range(2):
                e = e2 * 2 + d
                sacc = jnp.zeros((16,), jnp.float32)
                for k in range(D // 16):
                    l = bufL[b, e, pl.ds(k * 16, 16)]
                    r = bufR[b, e, pl.ds(k * 16, 16)]
                    z = l + r
                    z = jnp.maximum(z, 0.2 * z)
                    sacc = sacc + z * att_v[pl.ds(k * 16, 16)]
                alpha_sm[e] = jnp.sum(sacc)
            return 0
        lax.fori_loop(0, K // 2, _alpha, 0)

        def _scale(e, _):
            pv = jnp.exp(jnp.full((16,), alpha_sm[e], jnp.float32))
            for k in range(D // 16):
                bufL[b, e, pl.ds(k * 16, 16)] = bufL[b, e, pl.ds(k * 16, 16)] * pv
            stage_p[e, :] = pv
            return 0
        lax.fori_loop(0, K, _scale, 0)

        for g in range(K // 16):
            p16 = plsc.load_gather(stage_p, [g * 16 + iota16, iota16])
            dstv = idx_v[slot, 1, pl.ds(g * 16, 16)]
            plsc.addupdate_scatter(den_v, [zz16, dstv], p16)

        pltpu.sync_copy(bufL.at[b], accum_sh.at[idx_v.at[slot, 1]], add=True)

        @pl.when(j + 3 < NCH)
        def _():
            _idx_load(j + 3, slot)

    def _group(g, _):
        for u in range(6):
            _chunk(g * 6 + u, u)
        return 0
    lax.fori_loop(0, NCH // 6, _group, 0)

    plsc.subcore_barrier()
    pltpu.sync_copy(accum_sh.at[pl.ds(r0, RPT)],
                    out_acc.at[pl.ds(c * NP + r0, RPT)])
    pltpu.sync_copy(den_v, out_den.at[wid])


_gat_sc = functools.partial(
    pl.kernel,
    out_type=[
        jax.ShapeDtypeStruct((NC * NP, D), jnp.float32),
        jax.ShapeDtypeStruct((NW, 1, NP), jnp.float32),
    ],
    mesh=plsc.VectorSubcoreMesh(core_axis_name="c", subcore_axis_name="s"),
    compiler_params=pltpu.CompilerParams(needs_layout_passes=False),
    scratch_types=[
        pltpu.VMEM((3, 2, K), jnp.int32),
        pltpu.VMEM((2, K, D), jnp.float32),
        pltpu.VMEM((2, K, D), jnp.float32),
        pltpu.VMEM((K, 16), jnp.float32),
        pltpu.VMEM((1, NP), jnp.float32),
        pltpu.SMEM((K,), jnp.float32),
        pltpu.VMEM((D,), jnp.float32),
        pltpu.VMEM_SHARED((NP, D), jnp.float32),
        pltpu.SemaphoreType.DMA,
        pltpu.SemaphoreType.DMA,
        pltpu.SemaphoreType.DMA,
        pltpu.SemaphoreType.DMA,
        pltpu.SemaphoreType.DMA,
    ],
)(_gat_sc_body)


def _mm2_body(x_ref, wl_ref, wr_ref, xl_ref, xr_ref):
    xv = x_ref[...]
    xl_ref[...] = jnp.dot(xv, wl_ref[...], preferred_element_type=jnp.float32)
    xr_ref[...] = jnp.dot(xv, wr_ref[...], preferred_element_type=jnp.float32)


_mm2 = pl.pallas_call(
    _mm2_body,
    grid=(NP // BR,),
    in_specs=[
        pl.BlockSpec((BR, D), lambda i: (i, 0)),
        pl.BlockSpec((D, D), lambda i: (0, 0)),
        pl.BlockSpec((D, D), lambda i: (0, 0)),
    ],
    out_specs=[pl.BlockSpec((BR, D), lambda i: (i, 0))] * 2,
    out_shape=[jax.ShapeDtypeStruct((NP, D), jnp.float32)] * 2,
)


def _combine_mm_body(acc_ref, den_ref, b_ref, wl_ref, wr_ref, xl_ref, xr_ref):
    a = acc_ref[0] + acc_ref[1]
    d = jnp.sum(den_ref[...], axis=1)[:, None]
    h = jnp.maximum(a / jnp.maximum(d, TINY) + b_ref[...], 0.0)
    xl_ref[...] = jnp.dot(h, wl_ref[...], preferred_element_type=jnp.float32)
    xr_ref[...] = jnp.dot(h, wr_ref[...], preferred_element_type=jnp.float32)


_combine_mm = pl.pallas_call(
    _combine_mm_body,
    grid=(NP // BR,),
    in_specs=[
        pl.BlockSpec((NC, BR, D), lambda i: (0, i, 0)),
        pl.BlockSpec((BR, NW), lambda i: (i, 0)),
        pl.BlockSpec((1, D), lambda i: (0, 0)),
        pl.BlockSpec((D, D), lambda i: (0, 0)),
        pl.BlockSpec((D, D), lambda i: (0, 0)),
    ],
    out_specs=[pl.BlockSpec((BR, D), lambda i: (i, 0))] * 2,
    out_shape=[jax.ShapeDtypeStruct((NP, D), jnp.float32)] * 2,
)


def _head_body(acc_ref, den_ref, b_ref, batch_ref, linW_ref, linb_ref,
               outW_ref, outb_ref, o_ref):
    a = acc_ref[0] + acc_ref[1]
    d = jnp.sum(den_ref[...], axis=1)[:, None]
    h = jnp.maximum(a / jnp.maximum(d, TINY) + b_ref[...], 0.0)
    gids = lax.broadcasted_iota(jnp.int32, (G, NP), 0)
    onehot = (batch_ref[...] == gids).astype(jnp.float32)
    ssum = jnp.dot(onehot, h, preferred_element_type=jnp.float32)
    cnt = jnp.sum(onehot, axis=1, keepdims=True)
    pooled = ssum / jnp.maximum(cnt, 1.0)
    t = jnp.maximum(
        jnp.dot(pooled, linW_ref[...], preferred_element_type=jnp.float32)
        + linb_ref[...], 0.0)
    o_ref[...] = (jnp.dot(t, outW_ref[...], preferred_element_type=jnp.float32)
                  + outb_ref[...])


_head = pl.pallas_call(
    _head_body,
    out_shape=jax.ShapeDtypeStruct((G, 1), jnp.float32),
)


def kernel(x, edge_index, batch, Wl0, Wr0, att0, b0, Wl1, Wr1, att1, b1,
           Wl2, Wr2, att2, b2, lin_W, lin_b, out_W, out_b):
    f32 = jnp.float32
    x_p = jnp.zeros((NP, D), f32).at[:N].set(x)
    loops = jnp.arange(N, dtype=jnp.int32)
    pad = jnp.full((ETOT - E - N,), SENT, jnp.int32)
    srcp = jnp.concatenate([edge_index[0], loops, pad]).reshape(NW, NCH, 1, K)
    dstp = jnp.concatenate([edge_index[1], loops, pad]).reshape(NW, NCH, 1, K)
    eidx = jnp.concatenate([srcp, dstp], axis=2).reshape(-1)
    batch_p = jnp.concatenate(
        [batch, jnp.full((NP - N,), G, jnp.int32)]).reshape(1, NP)

    def layer(xl, xr, att):
        acc, den = _gat_sc(xl, xr, eidx, att)
        return acc.reshape(NC, NP, D), den.reshape(NW, NP).T

    xl, xr = _mm2(x_p, Wl0, Wr0)
    acc, den = layer(xl, xr, att0)
    xl, xr = _combine_mm(acc, den, b0.reshape(1, D), Wl1, Wr1)
    acc, den = layer(xl, xr, att1)
    xl, xr = _combine_mm(acc, den, b1.reshape(1, D), Wl2, Wr2)
    acc, den = layer(xl, xr, att2)
    return _head(acc, den, b2.reshape(1, D), batch_p,
                 lin_W, lin_b.reshape(1, D // 2), out_W, out_b.reshape(1, 1))
